# baseline (speedup 1.0000x reference)
import functools

import jax
import jax.numpy as jnp
from jax import lax
from jax.experimental import pallas as pl
from jax.experimental.pallas import tpu as pltpu
from jax.experimental.pallas import tpu_sc as plsc

_N = 50000
_E = 800000
_B = 512
_D = 78

_R = 50176
_WINW = 32
_NWIN = _R // _WINW
_KPAD = 768
_NSLOT = _NWIN * _KPAD
_KG = 192
_NSLOTP = _B * _KG
_SENT = _N

_F0 = 128
_F1 = 128
_F2 = 256
_F3 = 384

_NW = 32


def _pick_chunk(b_per_w, width):
    budget = max(8, (260000 // (width * 4)) // 8 * 8)
    c = 8
    for cand in range(8, budget + 1, 8):
        if b_per_w % cand == 0:
            c = cand
    return c


@functools.lru_cache(maxsize=None)
def _make_gather(nslot, width):
    b_per_w = nslot // _NW
    c = _pick_chunk(b_per_w, width)
    iters = b_per_w // c
    mesh = plsc.VectorSubcoreMesh(core_axis_name="c", subcore_axis_name="s")

    @functools.partial(
        pl.kernel,
        mesh=mesh,
        out_type=jax.ShapeDtypeStruct((nslot, width), jnp.float32),
        scratch_types=[
            pltpu.VMEM((c,), jnp.int32),
            pltpu.VMEM((c, width), jnp.float32),
            pltpu.SemaphoreType.DMA,
        ],
    )
    def gather_k(table_hbm, idx_hbm, out_hbm, idx_v, rows_v, sem):
        wid = lax.axis_index("s") * 2 + lax.axis_index("c")
        base = wid * b_per_w

        @pl.loop(0, iters)
        def _(i):
            off = base + i * c
            pltpu.sync_copy(idx_hbm.at[pl.ds(off, c)], idx_v)
            pltpu.async_copy(table_hbm.at[idx_v], rows_v, sem).wait()
            pltpu.sync_copy(rows_v, out_hbm.at[pl.ds(off, c)])

    return gather_k


def _sc_gather(table, idx):
    return _make_gather(idx.shape[0], table.shape[1])(table, idx)


def _mmscale_body(x_ref, w_ref, dv_ref, o_ref):
    o_ref[...] = dv_ref[...] * jnp.dot(
        x_ref[...], w_ref[...], preferred_element_type=jnp.float32
    )


def _mmscale(x, w, dinv):
    fin, fout = w.shape
    grid = _R // 512
    return pl.pallas_call(
        _mmscale_body,
        grid=(grid,),
        in_specs=[
            pl.BlockSpec((512, fin), lambda i: (i, 0)),
            pl.BlockSpec((fin, fout), lambda i: (0, 0)),
            pl.BlockSpec((512, 1), lambda i: (i, 0)),
        ],
        out_specs=pl.BlockSpec((512, fout), lambda i: (i, 0)),
        out_shape=jax.ShapeDtypeStruct((_R, fout), jnp.float32),
    )(x, w, dinv)


_WB = 4


def _segfin_body(msg_ref, dl_ref, g_ref, dv_ref, b_ref, o_ref):
    msg = msg_ref[...]
    dl = dl_ref[...][..., 0]
    segs = []
    for w in range(_WB):
        oh = (dl[w][:, None] == lax.broadcasted_iota(
            jnp.int32, (_KPAD, _WINW), 1)).astype(jnp.float32)
        segs.append(lax.dot_general(
            oh, msg[w], (((0,), (0,)), ((), ())),
            preferred_element_type=jnp.float32))
    seg = jnp.concatenate(segs, axis=0)
    dv = dv_ref[...]
    pre = dv * (seg + g_ref[...]) + b_ref[...]
    o_ref[...] = jnp.where((pre > 0.0) & (dv > 0.0), pre, 0.0)


def _segfin(msg, dstloc, g, dinv, b):
    f = msg.shape[-1]
    rows = _WB * _WINW
    grid = _NWIN // _WB
    return pl.pallas_call(
        _segfin_body,
        grid=(grid,),
        in_specs=[
            pl.BlockSpec((_WB, _KPAD, f), lambda i: (i, 0, 0)),
            pl.BlockSpec((_WB, _KPAD, 1), lambda i: (i, 0, 0)),
            pl.BlockSpec((rows, f), lambda i: (i, 0)),
            pl.BlockSpec((rows, 1), lambda i: (i, 0)),
            pl.BlockSpec((1, f), lambda i: (0, 0)),
        ],
        out_specs=pl.BlockSpec((rows, f), lambda i: (i, 0)),
        out_shape=jax.ShapeDtypeStruct((_R, f), jnp.float32),
    )(msg, dstloc[..., None], g, dinv, b)


def _poolmax_body(r_ref, o_ref):
    o_ref[...] = jnp.max(r_ref[...], axis=1)


def _poolmax(rows):
    f = rows.shape[-1]
    return pl.pallas_call(
        _poolmax_body,
        grid=(_B // 32,),
        in_specs=[pl.BlockSpec((32, _KG, f), lambda i: (i, 0, 0))],
        out_specs=pl.BlockSpec((32, f), lambda i: (i, 0)),
        out_shape=jax.ShapeDtypeStruct((_B, f), jnp.float32),
    )(rows)


def _head_body(p1_ref, p2_ref, cell_ref,
               wg1_ref, bg1_ref, wg2_ref, bg2_ref,
               wr1_ref, br1_ref, wr2_ref, br2_ref, wr3_ref, br3_ref,
               wf1_ref, bf1_ref, wf2_ref, bf2_ref, wo_ref, bo_ref,
               a_ref, o_ref):
    def mm(x, w):
        return jnp.dot(x, w, preferred_element_type=jnp.float32)

    def ghead(p):
        h = jnp.maximum(mm(p, wg1_ref[...]) + bg1_ref[...], 0.0)
        return mm(h, wg2_ref[...]) + bg2_ref[...]

    h1 = ghead(p1_ref[...])
    h2 = ghead(p2_ref[...])
    cv = jnp.maximum(mm(cell_ref[...], wr1_ref[...]) + br1_ref[...], 0.0)
    cv = jnp.maximum(mm(cv, wr2_ref[...]) + br2_ref[...], 0.0)
    cv = mm(cv, wr3_ref[...]) + br3_ref[...]
    xc = jnp.concatenate([h1, h2, cv], axis=1)
    nrm = jnp.sqrt(jnp.sum(xc * xc, axis=1, keepdims=True))
    xc = xc / jnp.maximum(nrm, 1e-12)
    a = a_ref[0, 0]
    xc = mm(xc, wf1_ref[...]) + bf1_ref[...]
    xc = jnp.where(xc > 0.0, xc, a * xc)
    xc = mm(xc, wf2_ref[...]) + bf2_ref[...]
    xc = jnp.where(xc > 0.0, xc, a * xc)
    o_ref[...] = mm(xc, wo_ref[...]) + bo_ref[...]


def _pad2(a, r, c):
    return jnp.pad(a, ((0, r - a.shape[0]), (0, c - a.shape[1])))


def _padb(b, c):
    return jnp.pad(b, (0, c - b.shape[0]))[None, :]


def _prep_edges(edge_index):
    src, dst = edge_index[0], edge_index[1]
    order = jnp.argsort(dst)
    dsts = dst[order]
    srcs = src[order]
    wptr = jnp.searchsorted(dsts, jnp.arange(_NWIN + 1, dtype=jnp.int32) * _WINW)
    nptr = jnp.searchsorted(dsts, jnp.arange(_N + 1, dtype=jnp.int32))
    deg = (nptr[1:] - nptr[:-1] + 1).astype(jnp.float32)
    dinv = lax.rsqrt(deg)
    dinv = jnp.pad(dinv, (0, _R - _N))[:, None]
    k = jnp.arange(_KPAD, dtype=jnp.int32)[None, :]
    pos = wptr[:-1, None].astype(jnp.int32) + k
    valid = pos < wptr[1:, None]
    posc = jnp.minimum(pos, _E - 1)
    psrc = jnp.where(valid, srcs[posc], _SENT).astype(jnp.int32)
    wbase = jnp.arange(_NWIN, dtype=jnp.int32)[:, None] * _WINW
    pdst = jnp.where(valid, dsts[posc] - wbase, 0).astype(jnp.int32)
    return psrc.reshape(-1), pdst, dinv


def _prep_pool(batch):
    gptr = jnp.searchsorted(batch, jnp.arange(_B + 1, dtype=jnp.int32))
    k = jnp.arange(_KG, dtype=jnp.int32)[None, :]
    pos = gptr[:-1, None].astype(jnp.int32) + k
    valid = pos < gptr[1:, None]
    return jnp.where(valid, jnp.minimum(pos, _N - 1), _SENT).astype(
        jnp.int32).reshape(-1)


def _branch(x, edge_index, batch, wc1, bc1, wc2, bc2, wc3, bc3):
    psrc, pdst, dinv = _prep_edges(edge_index)
    pnode = _prep_pool(batch)
    xp = _pad2(x, _R, _F0)

    g1 = _mmscale(xp, wc1, dinv)
    m1 = _sc_gather(g1, psrc).reshape(_NWIN, _KPAD, _F1)
    x1 = _segfin(m1, pdst, g1, dinv, bc1)

    g2 = _mmscale(x1, wc2, dinv)
    m2 = _sc_gather(g2, psrc).reshape(_NWIN, _KPAD, _F2)
    x2 = _segfin(m2, pdst, g2, dinv, bc2)

    g3 = _mmscale(x2, wc3, dinv)
    m3 = _sc_gather(g3, psrc).reshape(_NWIN, _KPAD, _F3)
    x3 = _segfin(m3, pdst, g3, dinv, bc3)

    rows = _sc_gather(x3, pnode).reshape(_B, _KG, _F3)
    return _poolmax(rows)


@jax.jit
def _run(x1, edge_index1, batch1, cell, x2, edge_index2, batch2,
         W_c1, b_c1, W_c2, b_c2, W_c3, b_c3, W_g1, b_g1, W_g2, b_g2,
         W_r1, b_r1, W_r2, b_r2, W_r3, b_r3, W_f1, b_f1, W_f2, b_f2,
         W_o, b_o, prelu_a):
    wc1 = _pad2(W_c1, _F0, _F1)
    wc2 = _pad2(W_c2, _F1, _F2)
    wc3 = _pad2(W_c3, _F2, _F3)
    bc1 = _padb(b_c1, _F1)
    bc2 = _padb(b_c2, _F2)
    bc3 = _padb(b_c3, _F3)

    p1 = _branch(x1, edge_index1, batch1, wc1, bc1, wc2, bc2, wc3, bc3)
    p2 = _branch(x2, edge_index2, batch2, wc1, bc1, wc2, bc2, wc3, bc3)

    cellp = _pad2(cell, _B, 1024)
    out = pl.pallas_call(
        _head_body,
        out_shape=jax.ShapeDtypeStruct((_B, 128), jnp.float32),
    )(p1, p2, cellp,
      _pad2(W_g1, _F3, 256), _padb(b_g1, 256),
      _pad2(W_g2, 256, 128), _padb(b_g2, 128),
      _pad2(W_r1, 1024, 512), _padb(b_r1, 512),
      W_r2, _padb(b_r2, 256),
      W_r3, _padb(b_r3, 128),
      W_f1, _padb(b_f1, 512),
      W_f2, _padb(b_f2, 128),
      _pad2(W_o, 128, 128), _padb(b_o, 128),
      prelu_a.reshape(1, 1))
    return out[:, :1]


def kernel(x1, edge_index1, batch1, cell, x2, edge_index2, batch2,
           W_c1, b_c1, W_c2, b_c2, W_c3, b_c3, W_g1, b_g1, W_g2, b_g2,
           W_r1, b_r1, W_r2, b_r2, W_r3, b_r3, W_f1, b_f1, W_f2, b_f2,
           W_o, b_o, prelu_a):
    return _run(x1, edge_index1, batch1, cell, x2, edge_index2, batch2,
                W_c1, b_c1, W_c2, b_c2, W_c3, b_c3, W_g1, b_g1, W_g2, b_g2,
                W_r1, b_r1, W_r2, b_r2, W_r3, b_r3, W_f1, b_f1, W_f2, b_f2,
                W_o, b_o, prelu_a)
